# baseline (speedup 1.0000x reference)
import functools

import numpy as np
import jax
import jax.numpy as jnp
from jax import lax
from jax.experimental import pallas as pl
from jax.experimental.pallas import tpu as pltpu
from jax.experimental.pallas import tpu_sc as plsc

_G, _N, _D, _K, _ITERS = 16, 2048, 4, 256, 3
_L = 16
_NCH = _N // _L
_KCH = _K // _L
_KP = _K + _L

_SEL_RANKS = np.round(np.linspace(0.0, _N - 1, _K)).astype(np.int32)
_RANK_LUT = np.full((_N,), -1, dtype=np.int32)
_RANK_LUT[_SEL_RANKS] = np.arange(_K, dtype=np.int32)


def _inv4(m):
    def det3(r, c):
        (r0, r1, r2), (c0, c1, c2) = r, c
        return (m[r0, c0] * (m[r1, c1] * m[r2, c2] - m[r1, c2] * m[r2, c1])
                - m[r0, c1] * (m[r1, c0] * m[r2, c2] - m[r1, c2] * m[r2, c0])
                + m[r0, c2] * (m[r1, c0] * m[r2, c1] - m[r1, c1] * m[r2, c0]))
    rows = [0, 1, 2, 3]
    cof = {}
    for i in range(4):
        for j in range(4):
            rr = tuple(r for r in rows if r != i)
            cc = tuple(c for c in rows if c != j)
            s = 1.0 if (i + j) % 2 == 0 else -1.0
            cof[i, j] = s * det3(rr, cc)
    det = (m[0, 0] * cof[0, 0] + m[0, 1] * cof[0, 1]
           + m[0, 2] * cof[0, 2] + m[0, 3] * cof[0, 3])
    inv_det = 1.0 / det
    return {(i, j): cof[j, i] * inv_det for i in range(4) for j in range(4)}


def _bf16r(x):
    b = plsc.bitcast(x, jnp.int32)
    r = b + jnp.int32(0x7FFF) + ((b >> jnp.int32(16)) & jnp.int32(1))
    return plsc.bitcast(r & jnp.int32(-65536), jnp.float32)


def _splat(ref, j):
    return plsc.load_gather(ref, [jnp.full((_L,), j, jnp.int32)])


_GATHER_DNUMS = lax.GatherDimensionNumbers(
    offset_dims=(), collapsed_slice_dims=(0,), start_index_map=(0,))


def _rotate(v, r):
    idx = ((lax.iota(jnp.int32, _L) + r) & (_L - 1)).reshape(_L, 1)
    return lax.gather(v, idx, dimension_numbers=_GATHER_DNUMS, slice_sizes=(1,),
                      mode=lax.GatherScatterMode.PROMISE_IN_BOUNDS)


def _allsum(v):
    for r in (8, 4, 2, 1):
        v = v + _rotate(v, r)
    return v


def _init_body(xt_hbm, lut_hbm, out_hbm, xv, xcv, dv, rkv, lutv,
               c0, c1, c2, c3, stg):
    cid = lax.axis_index("c")
    sid = lax.axis_index("s")

    @pl.when(cid == 0)
    def _run():
        g = sid
        pltpu.sync_copy(xt_hbm.at[g], xv)
        pltpu.sync_copy(lut_hbm, lutv)

        iota = lax.iota(jnp.int32, _L)
        fzero = jnp.zeros((_L,), jnp.float32)

        def mu_body(i, acc):
            sl = pl.ds(i * _L, _L)
            return tuple(acc[d] + xv[d, sl] for d in range(4))
        accs = lax.fori_loop(0, _NCH, mu_body, (fzero,) * 4)
        mu = [_allsum(a) * jnp.float32(1.0 / _N) for a in accs]

        def cen_body(i, _):
            sl = pl.ds(i * _L, _L)
            for d in range(4):
                xcv[d, sl] = xv[d, sl] - mu[d]
            return 0
        lax.fori_loop(0, _NCH, cen_body, 0)

        pairs = [(a, b) for a in range(4) for b in range(a, 4)]

        def cov_body(i, acc):
            sl = pl.ds(i * _L, _L)
            xc = [_bf16r(xcv[d, sl]) for d in range(4)]
            return tuple(acc[p] + xc[a] * xc[b] for p, (a, b) in enumerate(pairs))
        covs = lax.fori_loop(0, _NCH, cov_body, (fzero,) * len(pairs))
        sig = {}
        for p, (a, b) in enumerate(pairs):
            v = _allsum(covs[p])
            sig[a, b] = v
            sig[b, a] = v
        lam = _inv4(sig)
        lamb = {k: _bf16r(v) for k, v in lam.items()}

        def dist_body(i, _):
            sl = pl.ds(i * _L, _L)
            xc = [xcv[d, sl] for d in range(4)]
            xb = [_bf16r(v) for v in xc]
            y = [xb[0] * lamb[0, e] + xb[1] * lamb[1, e]
                 + xb[2] * lamb[2, e] + xb[3] * lamb[3, e] for e in range(4)]
            dv[sl] = y[0] * xc[0] + y[1] * xc[1] + y[2] * xc[2] + y[3] * xc[3]
            return 0
        lax.fori_loop(0, _NCH, dist_body, 0)

        def rank_body(ic, _):
            sl = pl.ds(ic * _L, _L)
            di = dv[sl]

            def le_body(j, r):
                return r + jnp.where(_splat(dv, j) <= di, 1, 0).astype(jnp.int32)
            r = lax.fori_loop(0, ic * _L, le_body, jnp.zeros((_L,), jnp.int32))

            def in_body(jl, r):
                dj = _splat(dv, ic * _L + jl)
                cond = (dj < di) | ((dj == di) & (iota > jl))
                return r + jnp.where(cond, 1, 0).astype(jnp.int32)
            r = lax.fori_loop(0, _L, in_body, r)

            def lt_body(j, r):
                return r + jnp.where(_splat(dv, j) < di, 1, 0).astype(jnp.int32)
            r = lax.fori_loop(ic * _L + _L, _N, lt_body, r)
            rkv[sl] = r
            return 0
        lax.fori_loop(0, _NCH, rank_body, 0)

        def sel_body(ic, _):
            sl = pl.ds(ic * _L, _L)
            slot = plsc.load_gather(lutv, [rkv[sl]])
            slot = jnp.where(slot >= 0, slot, _K).astype(jnp.int32)
            plsc.store_scatter(c0, [slot], xv[0, sl])
            plsc.store_scatter(c1, [slot], xv[1, sl])
            plsc.store_scatter(c2, [slot], xv[2, sl])
            plsc.store_scatter(c3, [slot], xv[3, sl])
            return 0
        lax.fori_loop(0, _NCH, sel_body, 0)

        for d, cd in enumerate((c0, c1, c2, c3)):
            def stage_body(kc, _):
                stg[pl.ds(kc * _L, _L)] = cd[pl.ds(kc * _L, _L)]
                return 0
            lax.fori_loop(0, _KCH, stage_body, 0)
            pltpu.sync_copy(stg, out_hbm.at[g, d])


def _step_body(xt_hbm, ci_hbm, out_hbm, xv, av, c0, c1, c2, c3, scv):
    cid = lax.axis_index("c")
    sid = lax.axis_index("s")

    @pl.when(cid == 0)
    def _run():
        g = sid
        pltpu.sync_copy(xt_hbm.at[g], xv)
        pltpu.sync_copy(ci_hbm.at[g, 0], c0)
        pltpu.sync_copy(ci_hbm.at[g, 1], c1)
        pltpu.sync_copy(ci_hbm.at[g, 2], c2)
        pltpu.sync_copy(ci_hbm.at[g, 3], c3)
        fzero = jnp.zeros((_L,), jnp.float32)

        def s_body(kc, _):
            sl = pl.ds(kc * _L, _L)
            a0, a1, a2, a3 = c0[sl], c1[sl], c2[sl], c3[sl]
            scv[sl] = a0 * a0 + a1 * a1 + a2 * a2 + a3 * a3
            return 0
        lax.fori_loop(0, _KCH, s_body, 0)

        BC = 4

        def asg_body(blk, _):
            base = blk * (BC * _L)
            xs = [[xv[d, pl.ds(base + t * _L, _L)] for d in range(4)]
                  for t in range(BC)]
            binf = jnp.full((_L,), jnp.inf, jnp.float32)
            bzero = jnp.zeros((_L,), jnp.int32)

            def k_body(k, carry):
                bests = carry[:BC]
                bidxs = carry[BC:]
                cc = (_splat(c0, k), _splat(c1, k), _splat(c2, k), _splat(c3, k))
                sk = _splat(scv, k)
                nb, ni = [], []
                for t in range(BC):
                    dot = (xs[t][0] * cc[0] + xs[t][1] * cc[1]
                           + xs[t][2] * cc[2] + xs[t][3] * cc[3])
                    dd = sk - jnp.float32(2.0) * dot
                    m = dd < bests[t]
                    nb.append(jnp.where(m, dd, bests[t]))
                    ni.append(jnp.where(m, k, bidxs[t]).astype(jnp.int32))
                return tuple(nb) + tuple(ni)
            carry = lax.fori_loop(0, _K, k_body, (binf,) * BC + (bzero,) * BC)
            for t in range(BC):
                av[pl.ds(base + t * _L, _L)] = carry[BC + t]
            return 0
        lax.fori_loop(0, _NCH // BC, asg_body, 0)

        def z_body(kc, _):
            sl = pl.ds(kc * _L, _L)
            c0[sl] = fzero
            c1[sl] = fzero
            c2[sl] = fzero
            c3[sl] = fzero
            scv[sl] = fzero
            return 0
        lax.fori_loop(0, _KCH, z_body, 0)

        ones = jnp.ones((_L,), jnp.float32)

        def acc_body(i, _):
            sl = pl.ds(i * _L, _L)
            a = av[sl]
            plsc.addupdate_scatter(c0, [a], _bf16r(xv[0, sl]))
            plsc.addupdate_scatter(c1, [a], _bf16r(xv[1, sl]))
            plsc.addupdate_scatter(c2, [a], _bf16r(xv[2, sl]))
            plsc.addupdate_scatter(c3, [a], _bf16r(xv[3, sl]))
            plsc.addupdate_scatter(scv, [a], ones)
            return 0
        lax.fori_loop(0, _NCH, acc_body, 0)

        def fin_body(kc, _):
            sl = pl.ds(kc * _L, _L)
            recip = jnp.float32(1.0) / jnp.maximum(scv[sl], jnp.float32(1.0))
            c0[sl] = c0[sl] * recip
            c1[sl] = c1[sl] * recip
            c2[sl] = c2[sl] * recip
            c3[sl] = c3[sl] * recip
            return 0
        lax.fori_loop(0, _KCH, fin_body, 0)

        pltpu.sync_copy(c0, out_hbm.at[g, 0])
        pltpu.sync_copy(c1, out_hbm.at[g, 1])
        pltpu.sync_copy(c2, out_hbm.at[g, 2])
        pltpu.sync_copy(c3, out_hbm.at[g, 3])


_MESH = plsc.VectorSubcoreMesh(core_axis_name="c", subcore_axis_name="s")

_init_kernel = functools.partial(
    pl.kernel,
    out_type=jax.ShapeDtypeStruct((_G, _D, _K), jnp.float32),
    mesh=_MESH,
    compiler_params=pltpu.CompilerParams(needs_layout_passes=False),
    scratch_types=[
        pltpu.VMEM((_D, _N), jnp.float32),
        pltpu.VMEM((_D, _N), jnp.float32),
        pltpu.VMEM((_N,), jnp.float32),
        pltpu.VMEM((_N,), jnp.int32),
        pltpu.VMEM((_N,), jnp.int32),
        pltpu.VMEM((_KP,), jnp.float32),
        pltpu.VMEM((_KP,), jnp.float32),
        pltpu.VMEM((_KP,), jnp.float32),
        pltpu.VMEM((_KP,), jnp.float32),
        pltpu.VMEM((_K,), jnp.float32),
    ],
)(_init_body)

_step_kernel = functools.partial(
    pl.kernel,
    out_type=jax.ShapeDtypeStruct((_G, _D, _K), jnp.float32),
    mesh=_MESH,
    compiler_params=pltpu.CompilerParams(needs_layout_passes=False),
    scratch_types=[
        pltpu.VMEM((_D, _N), jnp.float32),
        pltpu.VMEM((_N,), jnp.int32),
        pltpu.VMEM((_K,), jnp.float32),
        pltpu.VMEM((_K,), jnp.float32),
        pltpu.VMEM((_K,), jnp.float32),
        pltpu.VMEM((_K,), jnp.float32),
        pltpu.VMEM((_K,), jnp.float32),
    ],
)(_step_body)


def kernel(X):
    xt = jnp.transpose(X, (0, 2, 1))
    lut = jnp.asarray(_RANK_LUT)
    c = _init_kernel(xt, lut)
    for _ in range(_ITERS):
        c = _step_kernel(xt, c)
    return jnp.transpose(c, (0, 2, 1))


# speedup vs baseline: 2.9756x; 2.9756x over previous
import functools

import numpy as np
import jax
import jax.numpy as jnp
from jax import lax
from jax.experimental import pallas as pl
from jax.experimental.pallas import tpu as pltpu
from jax.experimental.pallas import tpu_sc as plsc

_G, _N, _D, _K, _ITERS = 16, 2048, 4, 256, 3
_L = 16
_NCH = _N // _L
_KCH = _K // _L
_KP = _K + _L

_SEL_RANKS = np.round(np.linspace(0.0, _N - 1, _K)).astype(np.int32)
_RANK_LUT = np.full((_N,), -1, dtype=np.int32)
_RANK_LUT[_SEL_RANKS] = np.arange(_K, dtype=np.int32)


def _inv4(m):
    def det3(r, c):
        (r0, r1, r2), (c0, c1, c2) = r, c
        return (m[r0, c0] * (m[r1, c1] * m[r2, c2] - m[r1, c2] * m[r2, c1])
                - m[r0, c1] * (m[r1, c0] * m[r2, c2] - m[r1, c2] * m[r2, c0])
                + m[r0, c2] * (m[r1, c0] * m[r2, c1] - m[r1, c1] * m[r2, c0]))
    rows = [0, 1, 2, 3]
    cof = {}
    for i in range(4):
        for j in range(4):
            rr = tuple(r for r in rows if r != i)
            cc = tuple(c for c in rows if c != j)
            s = 1.0 if (i + j) % 2 == 0 else -1.0
            cof[i, j] = s * det3(rr, cc)
    det = (m[0, 0] * cof[0, 0] + m[0, 1] * cof[0, 1]
           + m[0, 2] * cof[0, 2] + m[0, 3] * cof[0, 3])
    inv_det = 1.0 / det
    return {(i, j): cof[j, i] * inv_det for i in range(4) for j in range(4)}


def _bf16r(x):
    b = plsc.bitcast(x, jnp.int32)
    r = b + jnp.int32(0x7FFF) + ((b >> jnp.int32(16)) & jnp.int32(1))
    return plsc.bitcast(r & jnp.int32(-65536), jnp.float32)


def _splat(ref, j):
    return plsc.load_gather(ref, [jnp.full((_L,), j, jnp.int32)])


_GATHER_DNUMS = lax.GatherDimensionNumbers(
    offset_dims=(), collapsed_slice_dims=(0,), start_index_map=(0,))


def _rotate(v, r):
    idx = ((lax.iota(jnp.int32, _L) + r) & (_L - 1)).reshape(_L, 1)
    return lax.gather(v, idx, dimension_numbers=_GATHER_DNUMS, slice_sizes=(1,),
                      mode=lax.GatherScatterMode.PROMISE_IN_BOUNDS)


def _allsum(v):
    for r in (8, 4, 2, 1):
        v = v + _rotate(v, r)
    return v


def _init_body(xt_hbm, lut_hbm, out_hbm, xv, xcv, dv, rkv, lutv,
               c0, c1, c2, c3, stg):
    cid = lax.axis_index("c")
    sid = lax.axis_index("s")

    @pl.when(cid == 0)
    def _run():
        g = sid
        pltpu.sync_copy(xt_hbm.at[g], xv)
        pltpu.sync_copy(lut_hbm, lutv)

        iota = lax.iota(jnp.int32, _L)
        fzero = jnp.zeros((_L,), jnp.float32)

        def mu_body(i, acc):
            sl = pl.ds(i * _L, _L)
            return tuple(acc[d] + xv[d, sl] for d in range(4))
        accs = lax.fori_loop(0, _NCH, mu_body, (fzero,) * 4)
        mu = [_allsum(a) * jnp.float32(1.0 / _N) for a in accs]

        def cen_body(i, _):
            sl = pl.ds(i * _L, _L)
            for d in range(4):
                xcv[d, sl] = xv[d, sl] - mu[d]
            return 0
        lax.fori_loop(0, _NCH, cen_body, 0)

        pairs = [(a, b) for a in range(4) for b in range(a, 4)]

        def cov_body(i, acc):
            sl = pl.ds(i * _L, _L)
            xc = [_bf16r(xcv[d, sl]) for d in range(4)]
            return tuple(acc[p] + xc[a] * xc[b] for p, (a, b) in enumerate(pairs))
        covs = lax.fori_loop(0, _NCH, cov_body, (fzero,) * len(pairs))
        sig = {}
        for p, (a, b) in enumerate(pairs):
            v = _allsum(covs[p])
            sig[a, b] = v
            sig[b, a] = v
        lam = _inv4(sig)
        lamb = {k: _bf16r(v) for k, v in lam.items()}

        def dist_body(i, _):
            sl = pl.ds(i * _L, _L)
            xc = [xcv[d, sl] for d in range(4)]
            xb = [_bf16r(v) for v in xc]
            y = [xb[0] * lamb[0, e] + xb[1] * lamb[1, e]
                 + xb[2] * lamb[2, e] + xb[3] * lamb[3, e] for e in range(4)]
            dv[sl] = y[0] * xc[0] + y[1] * xc[1] + y[2] * xc[2] + y[3] * xc[3]
            return 0
        lax.fori_loop(0, _NCH, dist_body, 0)

        RB = 8

        def rank_body(ib, _):
            base = ib * (RB * _L)
            dis = [dv[pl.ds(base + t * _L, _L)] for t in range(RB)]
            fone = jnp.ones((_L,), jnp.float32)

            def le_body(j, accs):
                dj = _splat(dv, j)
                return tuple(a + jnp.where(dj <= dis[t], fone, fzero)
                             for t, a in enumerate(accs))
            accs = lax.fori_loop(0, base, le_body, (fzero,) * RB)

            def lt_body(j, accs):
                dj = _splat(dv, j)
                return tuple(a + jnp.where(dj < dis[t], fone, fzero)
                             for t, a in enumerate(accs))
            accs = lax.fori_loop(base + RB * _L, _N, lt_body, accs)

            def in_body(jl, accs):
                out = list(accs)
                for jc in range(RB):
                    dj = _splat(dv, base + jc * _L + jl)
                    for t in range(RB):
                        if jc < t:
                            cond = dj <= dis[t]
                        elif jc > t:
                            cond = dj < dis[t]
                        else:
                            cond = (dj < dis[t]) | ((dj == dis[t]) & (iota > jl))
                        out[t] = out[t] + jnp.where(cond, fone, fzero)
                return tuple(out)
            accs = lax.fori_loop(0, _L, in_body, accs)

            for t in range(RB):
                rkv[pl.ds(base + t * _L, _L)] = accs[t].astype(jnp.int32)
            return 0
        lax.fori_loop(0, _NCH // RB, rank_body, 0)

        def sel_body(ic, _):
            sl = pl.ds(ic * _L, _L)
            slot = plsc.load_gather(lutv, [rkv[sl]])
            slot = jnp.where(slot >= 0, slot, _K).astype(jnp.int32)
            plsc.store_scatter(c0, [slot], xv[0, sl])
            plsc.store_scatter(c1, [slot], xv[1, sl])
            plsc.store_scatter(c2, [slot], xv[2, sl])
            plsc.store_scatter(c3, [slot], xv[3, sl])
            return 0
        lax.fori_loop(0, _NCH, sel_body, 0)

        for d, cd in enumerate((c0, c1, c2, c3)):
            def stage_body(kc, _):
                stg[pl.ds(kc * _L, _L)] = cd[pl.ds(kc * _L, _L)]
                return 0
            lax.fori_loop(0, _KCH, stage_body, 0)
            pltpu.sync_copy(stg, out_hbm.at[g, d])


def _step_body(xt_hbm, ci_hbm, out_hbm, xv, av, c0, c1, c2, c3, scv):
    cid = lax.axis_index("c")
    sid = lax.axis_index("s")

    @pl.when(cid == 0)
    def _run():
        g = sid
        pltpu.sync_copy(xt_hbm.at[g], xv)
        pltpu.sync_copy(ci_hbm.at[g, 0], c0)
        pltpu.sync_copy(ci_hbm.at[g, 1], c1)
        pltpu.sync_copy(ci_hbm.at[g, 2], c2)
        pltpu.sync_copy(ci_hbm.at[g, 3], c3)
        fzero = jnp.zeros((_L,), jnp.float32)

        def s_body(kc, _):
            sl = pl.ds(kc * _L, _L)
            a0, a1, a2, a3 = c0[sl], c1[sl], c2[sl], c3[sl]
            scv[sl] = a0 * a0 + a1 * a1 + a2 * a2 + a3 * a3
            return 0
        lax.fori_loop(0, _KCH, s_body, 0)

        BC = 4

        def asg_body(blk, _):
            base = blk * (BC * _L)
            xs = [[xv[d, pl.ds(base + t * _L, _L)] for d in range(4)]
                  for t in range(BC)]
            binf = jnp.full((_L,), jnp.inf, jnp.float32)
            bzero = jnp.zeros((_L,), jnp.int32)

            def k_body(k, carry):
                bests = carry[:BC]
                bidxs = carry[BC:]
                cc = (_splat(c0, k), _splat(c1, k), _splat(c2, k), _splat(c3, k))
                sk = _splat(scv, k)
                nb, ni = [], []
                for t in range(BC):
                    dot = (xs[t][0] * cc[0] + xs[t][1] * cc[1]
                           + xs[t][2] * cc[2] + xs[t][3] * cc[3])
                    dd = sk - jnp.float32(2.0) * dot
                    m = dd < bests[t]
                    nb.append(jnp.where(m, dd, bests[t]))
                    ni.append(jnp.where(m, k, bidxs[t]).astype(jnp.int32))
                return tuple(nb) + tuple(ni)
            carry = lax.fori_loop(0, _K, k_body, (binf,) * BC + (bzero,) * BC)
            for t in range(BC):
                av[pl.ds(base + t * _L, _L)] = carry[BC + t]
            return 0
        lax.fori_loop(0, _NCH // BC, asg_body, 0)

        def z_body(kc, _):
            sl = pl.ds(kc * _L, _L)
            c0[sl] = fzero
            c1[sl] = fzero
            c2[sl] = fzero
            c3[sl] = fzero
            scv[sl] = fzero
            return 0
        lax.fori_loop(0, _KCH, z_body, 0)

        ones = jnp.ones((_L,), jnp.float32)

        def acc_body(i, _):
            sl = pl.ds(i * _L, _L)
            a = av[sl]
            plsc.addupdate_scatter(c0, [a], _bf16r(xv[0, sl]))
            plsc.addupdate_scatter(c1, [a], _bf16r(xv[1, sl]))
            plsc.addupdate_scatter(c2, [a], _bf16r(xv[2, sl]))
            plsc.addupdate_scatter(c3, [a], _bf16r(xv[3, sl]))
            plsc.addupdate_scatter(scv, [a], ones)
            return 0
        lax.fori_loop(0, _NCH, acc_body, 0)

        def fin_body(kc, _):
            sl = pl.ds(kc * _L, _L)
            recip = jnp.float32(1.0) / jnp.maximum(scv[sl], jnp.float32(1.0))
            c0[sl] = c0[sl] * recip
            c1[sl] = c1[sl] * recip
            c2[sl] = c2[sl] * recip
            c3[sl] = c3[sl] * recip
            return 0
        lax.fori_loop(0, _KCH, fin_body, 0)

        pltpu.sync_copy(c0, out_hbm.at[g, 0])
        pltpu.sync_copy(c1, out_hbm.at[g, 1])
        pltpu.sync_copy(c2, out_hbm.at[g, 2])
        pltpu.sync_copy(c3, out_hbm.at[g, 3])


_MESH = plsc.VectorSubcoreMesh(core_axis_name="c", subcore_axis_name="s")

_init_kernel = functools.partial(
    pl.kernel,
    out_type=jax.ShapeDtypeStruct((_G, _D, _K), jnp.float32),
    mesh=_MESH,
    compiler_params=pltpu.CompilerParams(needs_layout_passes=False),
    scratch_types=[
        pltpu.VMEM((_D, _N), jnp.float32),
        pltpu.VMEM((_D, _N), jnp.float32),
        pltpu.VMEM((_N,), jnp.float32),
        pltpu.VMEM((_N,), jnp.int32),
        pltpu.VMEM((_N,), jnp.int32),
        pltpu.VMEM((_KP,), jnp.float32),
        pltpu.VMEM((_KP,), jnp.float32),
        pltpu.VMEM((_KP,), jnp.float32),
        pltpu.VMEM((_KP,), jnp.float32),
        pltpu.VMEM((_K,), jnp.float32),
    ],
)(_init_body)

_step_kernel = functools.partial(
    pl.kernel,
    out_type=jax.ShapeDtypeStruct((_G, _D, _K), jnp.float32),
    mesh=_MESH,
    compiler_params=pltpu.CompilerParams(needs_layout_passes=False),
    scratch_types=[
        pltpu.VMEM((_D, _N), jnp.float32),
        pltpu.VMEM((_N,), jnp.int32),
        pltpu.VMEM((_K,), jnp.float32),
        pltpu.VMEM((_K,), jnp.float32),
        pltpu.VMEM((_K,), jnp.float32),
        pltpu.VMEM((_K,), jnp.float32),
        pltpu.VMEM((_K,), jnp.float32),
    ],
)(_step_body)


def kernel(X):
    xt = jnp.transpose(X, (0, 2, 1))
    lut = jnp.asarray(_RANK_LUT)
    c = _init_kernel(xt, lut)
    for _ in range(_ITERS):
        c = _step_kernel(xt, c)
    return jnp.transpose(c, (0, 2, 1))
